# baseline (speedup 1.0000x reference)
import jax
import jax.numpy as jnp
from jax import lax
from jax.experimental import pallas as pl
from jax.experimental.pallas import tpu as pltpu
from jax.experimental.pallas import tpu_sc as plsc

NC = 2
NS = 16
NW = NC * NS
B = 16
T = 2048
D = 128
B_TOT = B * T
PER_W = B_TOT // NW
C = 128
NCHUNK = PER_W // C
NBUF = 7
OLAG = 4


def _embd_body(wte_hbm, idx_hbm, wpe_hbm, out_hbm, idx_v, wpe_sh, *rest):
    bufs = rest[:NBUF]
    s_w = rest[NBUF:2 * NBUF]
    s_g = rest[2 * NBUF:3 * NBUF]
    s_o = rest[3 * NBUF:4 * NBUF]
    s_st = rest[4 * NBUF]
    del rest

    cid = lax.axis_index("c")
    sid = lax.axis_index("s")
    wid = sid * NC + cid
    base = wid * PER_W
    pos_base = cid * PER_W

    @pl.when(sid == 0)
    def _stage():
        pltpu.async_copy(wpe_hbm.at[pl.ds(pos_base, PER_W)], wpe_sh, s_st)

    pltpu.sync_copy(idx_hbm.at[pl.ds(base, PER_W)], idx_v)

    w_cp = [None] * NCHUNK
    g_cp = [None] * NCHUNK
    o_cp = [None] * NCHUNK
    o_waited = [False] * NCHUNK
    barriered = False
    for c in range(min(NBUF, NCHUNK)):
        w_cp[c] = pltpu.async_copy(
            wpe_hbm.at[pl.ds(pos_base + c * C, C)], bufs[c], s_w[c]
        )
    for step in range(NCHUNK + OLAG):
        c = step
        if c < NCHUNK:
            w_cp[c].wait()
            g_cp[c] = pltpu.async_copy(
                wte_hbm.at[idx_v.at[pl.ds(c * C, C)]],
                bufs[c % NBUF],
                s_g[c % NBUF],
                add=True,
            )
        co = step - OLAG
        if 0 <= co < NCHUNK:
            g_cp[co].wait()
            o_cp[co] = pltpu.async_copy(
                bufs[co % NBUF],
                out_hbm.at[pl.ds(base + co * C, C)],
                s_o[co % NBUF],
            )
        n = step + NBUF - OLAG - 1
        if NBUF <= n < NCHUNK:
            if not barriered:
                @pl.when(sid == 0)
                def _wait_stage():
                    pltpu.make_async_copy(
                        wpe_hbm.at[pl.ds(pos_base, PER_W)], wpe_sh, s_st
                    ).wait()
                plsc.subcore_barrier()
                barriered = True
            o_cp[n - NBUF].wait()
            o_waited[n - NBUF] = True
            w_cp[n] = pltpu.async_copy(
                wpe_sh.at[pl.ds(n * C, C)],
                bufs[n % NBUF],
                s_w[n % NBUF],
            )
    for c in range(NCHUNK):
        if not o_waited[c]:
            o_cp[c].wait()


def kernel(idx, wte, wpe):
    idx_flat = idx.reshape(-1).astype(jnp.int32)
    run = pl.kernel(
        _embd_body,
        out_type=jax.ShapeDtypeStruct((B_TOT, D), jnp.float32),
        mesh=plsc.VectorSubcoreMesh(core_axis_name="c", subcore_axis_name="s"),
        scratch_types=(
            [pltpu.VMEM((PER_W,), jnp.int32),
             pltpu.VMEM_SHARED((PER_W, D), jnp.float32)]
            + [pltpu.VMEM((C, D), jnp.float32) for _ in range(NBUF)]
            + [pltpu.SemaphoreType.DMA for _ in range(3 * NBUF + 1)]
        ),
    )
    out = run(wte, idx_flat, wpe)
    return out.reshape(B, T, D)


# speedup vs baseline: 1.2261x; 1.2261x over previous
import jax
import jax.numpy as jnp
from jax import lax
from jax.experimental import pallas as pl
from jax.experimental.pallas import tpu as pltpu
from jax.experimental.pallas import tpu_sc as plsc

NC = 2
NS = 16
NW = NC * NS
B = 16
T = 2048
D = 128
B_TOT = B * T
PER_W = B_TOT // NW
C = 128
NCHUNK = PER_W // C
NBUF = 7
OLAG = 4


def _embd_body(wte_hbm, idx_hbm, wpe_hbm, out_hbm, idx_v, wpe_sh, *rest):
    bufs = rest[:NBUF]
    s_w = rest[NBUF:2 * NBUF]
    s_g = rest[2 * NBUF:3 * NBUF]
    s_o = rest[3 * NBUF:4 * NBUF]
    s_st = rest[4 * NBUF]
    del rest

    cid = lax.axis_index("c")
    sid = lax.axis_index("s")
    wid = sid * NC + cid
    base = wid * PER_W
    pos_base = cid * PER_W

    @pl.when(sid == 0)
    def _stage():
        pltpu.async_copy(wpe_hbm.at[pl.ds(pos_base, PER_W)], wpe_sh, s_st)

    pltpu.sync_copy(idx_hbm.at[pl.ds(base, PER_W)], idx_v)

    @pl.when(sid == 0)
    def _wait_stage():
        pltpu.make_async_copy(
            wpe_hbm.at[pl.ds(pos_base, PER_W)], wpe_sh, s_st
        ).wait()

    plsc.subcore_barrier()

    w_cp = [None] * NCHUNK
    g_cp = [None] * NCHUNK
    o_cp = [None] * NCHUNK
    o_waited = [False] * NCHUNK
    for c in range(min(NBUF, NCHUNK)):
        w_cp[c] = pltpu.async_copy(
            wpe_sh.at[pl.ds(c * C, C)], bufs[c], s_w[c]
        )
    for step in range(NCHUNK + OLAG):
        c = step
        if c < NCHUNK:
            w_cp[c].wait()
            g_cp[c] = pltpu.async_copy(
                wte_hbm.at[idx_v.at[pl.ds(c * C, C)]],
                bufs[c % NBUF],
                s_g[c % NBUF],
                add=True,
            )
        co = step - OLAG
        if 0 <= co < NCHUNK:
            g_cp[co].wait()
            o_cp[co] = pltpu.async_copy(
                bufs[co % NBUF],
                out_hbm.at[pl.ds(base + co * C, C)],
                s_o[co % NBUF],
            )
        n = step + NBUF - OLAG - 1
        if NBUF <= n < NCHUNK:
            o_cp[n - NBUF].wait()
            o_waited[n - NBUF] = True
            w_cp[n] = pltpu.async_copy(
                wpe_sh.at[pl.ds(n * C, C)],
                bufs[n % NBUF],
                s_w[n % NBUF],
            )
    for c in range(NCHUNK):
        if not o_waited[c]:
            o_cp[c].wait()


def kernel(idx, wte, wpe):
    idx_flat = idx.reshape(-1).astype(jnp.int32)
    run = pl.kernel(
        _embd_body,
        out_type=jax.ShapeDtypeStruct((B_TOT, D), jnp.float32),
        mesh=plsc.VectorSubcoreMesh(core_axis_name="c", subcore_axis_name="s"),
        scratch_types=(
            [pltpu.VMEM((PER_W,), jnp.int32),
             pltpu.VMEM_SHARED((PER_W, D), jnp.float32)]
            + [pltpu.VMEM((C, D), jnp.float32) for _ in range(NBUF)]
            + [pltpu.SemaphoreType.DMA for _ in range(3 * NBUF + 1)]
        ),
    )
    out = run(wte, idx_flat, wpe)
    return out.reshape(B, T, D)
